# baseline (speedup 1.0000x reference)
import functools
import jax
import jax.numpy as jnp
from jax.experimental import pallas as pl
from jax.experimental.pallas import tpu as pltpu

DIM = 1024
MOTIF = 268
MOTIF_PAD = 384
HEADS = 16
DH = 64
E = 8
K = 2
HID = 1024
S = 2048
TM = 256
NT = S // TM
NEG = -1e30


def _ln_tile(x, s, b):
    m = jnp.mean(x, axis=-1, keepdims=True)
    v = jnp.mean((x - m) ** 2, axis=-1, keepdims=True)
    return (x - m) * jax.lax.rsqrt(v + 1e-5) * s + b


def _k1_body(x_ref, s_ref, b_ref, w_ref, wb_ref, out_ref):
    x = _ln_tile(x_ref[...], s_ref[...], b_ref[...])
    out_ref[...] = jnp.dot(x, w_ref[...], preferred_element_type=jnp.float32) + wb_ref[...]


def _k1(x, ln1_s, ln1_b, qkv_w, qkv_b):
    return pl.pallas_call(
        _k1_body,
        grid=(NT, 3),
        in_specs=[
            pl.BlockSpec((TM, DIM), lambda i, j: (i, 0)),
            pl.BlockSpec((1, DIM), lambda i, j: (0, 0)),
            pl.BlockSpec((1, DIM), lambda i, j: (0, 0)),
            pl.BlockSpec((DIM, DIM), lambda i, j: (0, j)),
            pl.BlockSpec((1, DIM), lambda i, j: (0, j)),
        ],
        out_specs=pl.BlockSpec((TM, DIM), lambda i, j: (i, j)),
        out_shape=jax.ShapeDtypeStruct((S, 3 * DIM), jnp.float32),
    )(x, ln1_s, ln1_b, qkv_w, qkv_b)


def _k2_body(q_ref, k_ref, v_ref, o_ref, *, scale):
    q = q_ref[0]
    k = k_ref[0]
    s = jnp.dot(q, k.T, preferred_element_type=jnp.float32) * scale
    m = jnp.max(s, axis=-1, keepdims=True)
    p = jnp.exp(s - m)
    p = p / jnp.sum(p, axis=-1, keepdims=True)
    o_ref[0] = jnp.dot(p, v_ref[0], preferred_element_type=jnp.float32)


def _k2(qkv3):
    return pl.pallas_call(
        functools.partial(_k2_body, scale=DIM ** -0.5),
        grid=(HEADS, NT),
        in_specs=[
            pl.BlockSpec((1, TM, DH), lambda h, i: (3 * h, i, 0)),
            pl.BlockSpec((1, S, DH), lambda h, i: (3 * h + 1, 0, 0)),
            pl.BlockSpec((1, S, DH), lambda h, i: (3 * h + 2, 0, 0)),
        ],
        out_specs=pl.BlockSpec((1, TM, DH), lambda h, i: (h, i, 0)),
        out_shape=jax.ShapeDtypeStruct((HEADS, S, DH), jnp.float32),
    )(qkv3, qkv3, qkv3)


def _k3_body(o_ref, pw_ref, pb_ref, ss_ref, sb_ref, xin_ref, l2s_ref, l2b_ref,
             wg_ref, ao_ref, xf_ref, gates_ref, imp_ref, load_ref, loss_ref):
    i = pl.program_id(0)
    o = jnp.dot(o_ref[...], pw_ref[...], preferred_element_type=jnp.float32) + pb_ref[...]
    o = o * ss_ref[...] + sb_ref[...]
    ao = o + xin_ref[...]
    ao_ref[...] = ao
    xf = _ln_tile(ao, l2s_ref[...], l2b_ref[...])
    xf_ref[...] = xf
    lg = jnp.dot(xf, wg_ref[...], preferred_element_type=jnp.float32)
    col = jax.lax.broadcasted_iota(jnp.int32, (TM, 128), 1)
    lg = jnp.where(col < E, lg, NEG)
    m1 = jnp.max(lg, axis=1, keepdims=True)
    i1 = jnp.min(jnp.where(lg == m1, col, 128), axis=1, keepdims=True)
    lg2 = jnp.where(col == i1, NEG, lg)
    m2 = jnp.max(lg2, axis=1, keepdims=True)
    i2 = jnp.min(jnp.where(lg2 == m2, col, 128), axis=1, keepdims=True)
    e2 = jnp.exp(m2 - m1)
    g1 = 1.0 / (1.0 + e2)
    g2 = e2 / (1.0 + e2)
    gates = jnp.where(col == i1, g1, 0.0) + jnp.where(col == i2, g2, 0.0)
    gates_ref[...] = gates
    imp_t = jnp.sum(gates, axis=0, keepdims=True)
    load_t = jnp.sum((gates > 0.0).astype(jnp.float32), axis=0, keepdims=True)

    @pl.when(i == 0)
    def _():
        imp_ref[...] = imp_t
        load_ref[...] = load_t

    @pl.when(i > 0)
    def _():
        imp_ref[...] = imp_ref[...] + imp_t
        load_ref[...] = load_ref[...] + load_t

    @pl.when(i == pl.num_programs(0) - 1)
    def _():
        lane = jax.lax.broadcasted_iota(jnp.int32, (1, 128), 1)
        valid = lane < E

        def cv(v):
            v = jnp.where(valid, v, 0.0)
            mean = jnp.sum(v) / E
            var = jnp.sum(jnp.where(valid, (v - mean) ** 2, 0.0)) / (E - 1)
            return var / (mean * mean + 1e-10)

        loss_ref[...] = ((cv(imp_ref[...]) + cv(load_ref[...])) * 0.01).reshape(1, 1)


def _k3(o, attn_pw, attn_pb, attn_ss, attn_sb, x_in, ln2_s, ln2_b, wg_pad):
    return pl.pallas_call(
        _k3_body,
        grid=(NT,),
        in_specs=[
            pl.BlockSpec((TM, DIM), lambda i: (i, 0)),
            pl.BlockSpec((DIM, DIM), lambda i: (0, 0)),
            pl.BlockSpec((1, DIM), lambda i: (0, 0)),
            pl.BlockSpec((1, DIM), lambda i: (0, 0)),
            pl.BlockSpec((1, DIM), lambda i: (0, 0)),
            pl.BlockSpec((TM, DIM), lambda i: (i, 0)),
            pl.BlockSpec((1, DIM), lambda i: (0, 0)),
            pl.BlockSpec((1, DIM), lambda i: (0, 0)),
            pl.BlockSpec((DIM, 128), lambda i: (0, 0)),
        ],
        out_specs=[
            pl.BlockSpec((TM, DIM), lambda i: (i, 0)),
            pl.BlockSpec((TM, DIM), lambda i: (i, 0)),
            pl.BlockSpec((TM, 128), lambda i: (i, 0)),
            pl.BlockSpec((1, 128), lambda i: (0, 0)),
            pl.BlockSpec((1, 128), lambda i: (0, 0)),
            pl.BlockSpec((1, 1), lambda i: (0, 0)),
        ],
        out_shape=[
            jax.ShapeDtypeStruct((S, DIM), jnp.float32),
            jax.ShapeDtypeStruct((S, DIM), jnp.float32),
            jax.ShapeDtypeStruct((S, 128), jnp.float32),
            jax.ShapeDtypeStruct((1, 128), jnp.float32),
            jax.ShapeDtypeStruct((1, 128), jnp.float32),
            jax.ShapeDtypeStruct((1, 1), jnp.float32),
        ],
    )(o, attn_pw, attn_pb, attn_ss, attn_sb, x_in, ln2_s, ln2_b, wg_pad)


def _k4_body(xf_ref, w1_ref, b1_ref, w2_ref, b2_ref, g_ref, y_ref):
    e = pl.program_id(1)
    x = xf_ref[...]
    h = jnp.dot(x, w1_ref[0], preferred_element_type=jnp.float32) + b1_ref[0]
    h = jax.nn.gelu(h)
    ye = jnp.dot(h, w2_ref[0], preferred_element_type=jnp.float32) + b2_ref[0]
    col = jax.lax.broadcasted_iota(jnp.int32, (TM, 128), 1)
    g = jnp.sum(jnp.where(col == e, g_ref[...], 0.0), axis=1, keepdims=True)
    contrib = ye * g

    @pl.when(e == 0)
    def _():
        y_ref[...] = contrib

    @pl.when(e > 0)
    def _():
        y_ref[...] = y_ref[...] + contrib


def _k4_dense(xf, ew1, eb1, ew2, eb2, gates):
    return pl.pallas_call(
        _k4_body,
        grid=(NT, E),
        in_specs=[
            pl.BlockSpec((TM, DIM), lambda i, e: (i, 0)),
            pl.BlockSpec((1, DIM, HID), lambda i, e: (e, 0, 0)),
            pl.BlockSpec((1, 1, HID), lambda i, e: (e, 0, 0)),
            pl.BlockSpec((1, HID, DIM), lambda i, e: (e, 0, 0)),
            pl.BlockSpec((1, 1, DIM), lambda i, e: (e, 0, 0)),
            pl.BlockSpec((TM, 128), lambda i, e: (i, 0)),
        ],
        out_specs=pl.BlockSpec((TM, DIM), lambda i, e: (i, 0)),
        out_shape=jax.ShapeDtypeStruct((S, DIM), jnp.float32),
    )(xf, ew1, eb1.reshape(E, 1, HID), ew2, eb2.reshape(E, 1, DIM), gates)


def _k7_body(y_ref, ss_ref, sb_ref, ao_ref, pw_ref, pb_ref, out_ref):
    z = y_ref[...] * ss_ref[...] + sb_ref[...] + ao_ref[...]
    out_ref[...] = jnp.dot(z, pw_ref[...], preferred_element_type=jnp.float32) + pb_ref[...]


def _k7(y, mlp_ss, mlp_sb, ao, pw_pad, pb_pad):
    return pl.pallas_call(
        _k7_body,
        grid=(NT,),
        in_specs=[
            pl.BlockSpec((TM, DIM), lambda i: (i, 0)),
            pl.BlockSpec((1, DIM), lambda i: (0, 0)),
            pl.BlockSpec((1, DIM), lambda i: (0, 0)),
            pl.BlockSpec((TM, DIM), lambda i: (i, 0)),
            pl.BlockSpec((DIM, MOTIF_PAD), lambda i: (0, 0)),
            pl.BlockSpec((1, MOTIF_PAD), lambda i: (0, 0)),
        ],
        out_specs=pl.BlockSpec((TM, MOTIF_PAD), lambda i: (i, 0)),
        out_shape=jax.ShapeDtypeStruct((S, MOTIF_PAD), jnp.float32),
    )(y, mlp_ss, mlp_sb, ao, pw_pad, pb_pad)


def kernel(inputs, ln1_s, ln1_b, qkv_w, qkv_b, attn_pw, attn_pb, attn_ss, attn_sb,
           ln2_s, ln2_b, w_gate, ew1, eb1, ew2, eb2, mlp_ss, mlp_sb, proj_w, proj_b):
    x = inputs.reshape(S, DIM)
    r1 = lambda a: a.reshape(1, -1)

    qkv = _k1(x, r1(ln1_s), r1(ln1_b), qkv_w, r1(qkv_b))
    qkv3 = qkv.reshape(S, 3 * HEADS, DH).transpose(1, 0, 2)
    o3 = _k2(qkv3)
    o = o3.transpose(1, 0, 2).reshape(S, DIM)
    wg_pad = jnp.pad(w_gate, ((0, 0), (0, 128 - E)))
    ao, xf, gates, imp, load, loss = _k3(
        o, attn_pw, r1(attn_pb), r1(attn_ss), r1(attn_sb), x,
        r1(ln2_s), r1(ln2_b), wg_pad)
    y = _k4_dense(xf, ew1, eb1, ew2, eb2, gates)
    pw_pad = jnp.pad(proj_w, ((0, 0), (0, MOTIF_PAD - MOTIF)))
    pb_pad = jnp.pad(proj_b, (0, MOTIF_PAD - MOTIF)).reshape(1, MOTIF_PAD)
    out = _k7(y, r1(mlp_ss), r1(mlp_sb), ao, pw_pad, pb_pad)
    return out[:, :MOTIF].reshape(1, S, MOTIF), loss.reshape(())
